# baseline (speedup 1.0000x reference)
import dataclasses
import functools

import jax
import jax.numpy as jnp
from jax import lax
from jax.experimental import pallas as pl
from jax.experimental.pallas import tpu as pltpu
from jax.experimental.pallas import tpu_sc as plsc

N_NODES = 10000
INPUT_DIM = 512
OUTPUT_DIM = 256
N_LAYERS = 8
LAYER = N_NODES // N_LAYERS
N_EDGES = 160000
BATCH = 64

NC = 2
NS = 16
LANES = 16
HB = BATCH // NC

EPT = N_EDGES // NS
CHUNK = 128
RAW_CHUNKS = -(-EPT // CHUNK)
EPT_PAD = RAW_CHUNKS * CHUNK
RAW_VECS = EPT_PAD // LANES
N_BINS = N_LAYERS - 1
NQ = 4
QUART_VECS = RAW_VECS // NQ
QUART_EDGES = EPT_PAD // NQ
BIN_CHUNKS = -(-(EPT_PAD + NQ * N_BINS * (CHUNK - 1)) // CHUNK) + 1

ACC_REAL = N_NODES - LAYER
ACC_PT = 552
ACC_ROWS = ACC_PT * NS
DUMMY0 = 8752
H_ROWS = 8832
ACT_PT = 80
INIT_PT = 48
LOG0 = ACC_REAL - OUTPUT_DIM

_mesh = plsc.VectorSubcoreMesh(core_axis_name="c", subcore_axis_name="s")

_cp = pltpu.CompilerParams()
for _f, _v in (("needs_layout_passes", False),
               ("use_tc_tiling_on_sc", False)):
    if _f in pltpu.CompilerParams.__dataclass_fields__:
        _cp = dataclasses.replace(_cp, **{_f: _v})


def _key(d):
    return lax.shift_right_logical(d * 6711, 23)


def _act_block(a, code, wv):
    e0 = jnp.exp(-a)
    sig = 1.0 / (1.0 + e0)
    e2 = e0 * e0
    tnh = 2.0 / (1.0 + e2) - 1.0
    rel = jnp.maximum(a, 0.0)
    cb = jnp.full((LANES,), code, dtype=jnp.int32)
    h = jnp.where(cb == 1, sig, a)
    h = jnp.where(cb == 2, rel, h)
    h = jnp.where(cb == 3, tnh, h)
    return h * wv


@functools.partial(
    pl.kernel,
    out_type=[
        jax.ShapeDtypeStruct((NC, OUTPUT_DIM, HB), jnp.float32),
        jax.ShapeDtypeStruct((NC, H_ROWS, HB), jnp.float32),
    ],
    mesh=_mesh,
    scratch_types=[
        pltpu.VMEM_SHARED((ACC_ROWS, HB), jnp.float32),
        pltpu.VMEM((EPT_PAD,), jnp.int32),
        pltpu.VMEM((EPT_PAD,), jnp.int32),
        pltpu.VMEM((BIN_CHUNKS, CHUNK), jnp.int32),
        pltpu.VMEM((BIN_CHUNKS, CHUNK), jnp.int32),
        pltpu.VMEM((8, CHUNK, HB), jnp.float32),
        pltpu.VMEM((ACT_PT, HB), jnp.float32),
        pltpu.VMEM((64, HB), jnp.float32),
        pltpu.VMEM((32, HB), jnp.float32),
        pltpu.VMEM((7552,), jnp.int32),
        pltpu.VMEM((784,), jnp.int32),
        pltpu.VMEM((LANES,), jnp.float32),
        pltpu.VMEM((LANES,), jnp.int32),
        pltpu.VMEM((LANES,), jnp.int32),
        pltpu.VMEM((LANES,), jnp.int32),
        pltpu.VMEM((LANES,), jnp.int32),
        pltpu.SMEM((32,), jnp.int32),
        pltpu.SMEM((32,), jnp.int32),
    ] + [pltpu.SemaphoreType.DMA] * 16,
    compiler_params=_cp,
)
def _sc_forward(x3, w16, esrc, edst, codes, logt, hout,
                acc, rsrc, rdst, bsrc, bdst, gbuf, abuf, zbuf, xbuf,
                cab, cib, wbuf, curvA, curvB, curvC, curvD,
                starts2, nch2, *sems):
    gsems = sems[:8]
    ssems = sems[8:]
    cid = lax.axis_index("c")
    sid = lax.axis_index("s")
    hc = hout.at[cid]

    _scope_p0 = jax.named_scope("p0_stage")
    _scope_p0.__enter__()
    x_src = x3.at[cid].at[pl.ds(sid * 32, 32)]
    s_src = esrc.at[pl.ds(sid * EPT, EPT)]
    d_src = edst.at[pl.ds(sid * EPT, EPT)]
    ca_src = codes.at[pl.ds(LAYER - 2, 7552)]
    ci_src = codes.at[pl.ds(INPUT_DIM, 784)]
    pltpu.async_copy(s_src, rsrc.at[pl.ds(0, EPT)], gsems[0])
    pltpu.async_copy(d_src, rdst.at[pl.ds(0, EPT)], gsems[1])
    pltpu.async_copy(ca_src, cab, gsems[2])
    pltpu.async_copy(ci_src, cib, gsems[3])
    pltpu.async_copy(x_src, xbuf, gsems[4])
    pltpu.sync_copy(w16, wbuf)
    wv = wbuf[...]

    iota = lax.iota(jnp.int32, LANES)

    @pl.loop(0, (EPT_PAD - EPT) // LANES)
    def _(q):
        v = iota + q * LANES
        rsrc[pl.ds(EPT + q * LANES, LANES)] = v & 511
        rdst[pl.ds(EPT + q * LANES, LANES)] = (LAYER + DUMMY0) + (v & 63)

    @pl.loop(0, 64)
    def _(r):
        zbuf[r, pl.ds(0, 16)] = jnp.zeros((16,), jnp.float32)
        zbuf[r, pl.ds(16, 16)] = jnp.zeros((16,), jnp.float32)

    @pl.loop(0, 8)
    def _(k):
        pltpu.sync_copy(zbuf, acc.at[pl.ds(sid * ACC_PT + k * 64, 64)])
    pltpu.sync_copy(zbuf.at[pl.ds(0, 40)],
                    acc.at[pl.ds(sid * ACC_PT + 512, 40)])

    pltpu.make_async_copy(x_src, xbuf, gsems[4]).wait()

    @pl.loop(0, 32)
    def _(r):
        xbuf[r, pl.ds(0, 16)] = xbuf[r, pl.ds(0, 16)] * wv
        xbuf[r, pl.ds(16, 16)] = xbuf[r, pl.ds(16, 16)] * wv
    pltpu.sync_copy(xbuf, hc.at[pl.ds(sid * 32, 32)])

    w_s = wv[0]
    pltpu.make_async_copy(ci_src, cib, gsems[3]).wait()

    @pl.loop(0, INIT_PT // 8)
    def _(ch):
        cv = cib[pl.ds(sid * INIT_PT + ch * 8, 16)]
        for r in range(8):
            row = ch * 8 + r
            val = jnp.where(cv[r] == 1, 0.5 * w_s, 0.0)
            abuf[row, pl.ds(0, 16)] = jnp.full((16,), val, jnp.float32)
            abuf[row, pl.ds(16, 16)] = jnp.full((16,), val, jnp.float32)
    pltpu.sync_copy(abuf.at[pl.ds(0, INIT_PT)],
                    hc.at[pl.ds(INPUT_DIM + sid * INIT_PT, INIT_PT)])

    _scope_p0.__exit__(None, None, None)

    _scope_pf = jax.named_scope("p1_prefill")
    _scope_pf.__enter__()

    @pl.loop(0, BIN_CHUNKS * CHUNK // LANES)
    def _prefill(i):
        jj = i // (CHUNK // LANES)
        qq = i % (CHUNK // LANES)
        v = iota + i * LANES
        bsrc[jj, pl.ds(qq * 16, 16)] = v & 511
        bdst[jj, pl.ds(qq * 16, 16)] = DUMMY0 + (v & 63)

    _scope_pf.__exit__(None, None, None)
    _scope_ct = jax.named_scope("p1_count")
    _scope_ct.__enter__()
    pltpu.make_async_copy(d_src, rdst.at[pl.ds(0, EPT)], gsems[1]).wait()
    pltpu.make_async_copy(s_src, rsrc.at[pl.ds(0, EPT)], gsems[0]).wait()
    pltpu.make_async_copy(ca_src, cab, gsems[2]).wait()
    zv = jnp.zeros((LANES,), jnp.int32)

    def _count_pair(q0):
        def body(i, accs):
            kA = _key(rdst[pl.ds(q0 * QUART_EDGES + i * LANES, LANES)]
                      - LAYER)
            kB = _key(rdst[pl.ds((q0 + 1) * QUART_EDGES + i * LANES, LANES)]
                      - LAYER)
            return (tuple(accs[l] + (kA == l).astype(jnp.int32)
                          for l in range(N_BINS)) +
                    tuple(accs[N_BINS + l] + (kB == l).astype(jnp.int32)
                          for l in range(N_BINS)))
        return lax.fori_loop(0, QUART_VECS, body, (zv,) * (2 * N_BINS))

    accs = _count_pair(0) + _count_pair(2)

    prev = jnp.int32(0)
    for t in range(NQ * N_BINS):
        q, l = t // N_BINS, t % N_BINS
        sl = 8 * q + l
        n = (jnp.sum(accs[t]) + CHUNK - 1) // CHUNK
        nch2[sl] = n
        starts2[sl] = prev
        prev = prev + n * CHUNK

    _scope_ct.__exit__(None, None, None)
    _scope_di = jax.named_scope("p1_dist")
    _scope_di.__enter__()

    curvs = (curvA, curvB, curvC, curvD)
    for q in range(NQ):
        cur = zv
        for l in range(N_BINS):
            cur = jnp.where(iota == l, starts2[8 * q + l], cur)
        curvs[q][...] = cur

    def _dist_body(i, carry):
        for q in range(NQ):
            off = q * QUART_EDGES + i * LANES
            s = rsrc[pl.ds(off, LANES)]
            dm = rdst[pl.ds(off, LANES)] - LAYER
            k = _key(dm)
            real = k < N_BINS
            cnt, last = plsc.scan_count(k, mask=real)
            base = plsc.load_gather(curvs[q], [k])
            pos = base + cnt - 1
            hi = lax.shift_right_logical(pos, 7)
            lo = pos & (CHUNK - 1)
            plsc.store_scatter(bsrc, [hi, lo], s, mask=real)
            plsc.store_scatter(bdst, [hi, lo], dm, mask=real)
            plsc.addupdate_scatter(curvs[q], [k], cnt, mask=last)
        return carry

    lax.fori_loop(0, QUART_VECS, _dist_body, jnp.int32(0))

    _scope_di.__exit__(None, None, None)

    plsc.subcore_barrier()

    @pl.loop(0, N_BINS)
    def _(l):
        _scope_ed = jax.named_scope("p2_edges")
        _scope_ed.__enter__()
        cbs = [starts2[8 * q + l] // CHUNK for q in range(NQ)]
        nqs = [nch2[8 * q + l] for q in range(NQ)]
        n01 = nqs[0] + nqs[1]
        n012 = n01 + nqs[2]
        nchl = n012 + nqs[3]
        ngrp = (nchl + 7) // 8

        def _cidx(j):
            r = cbs[3] + (j - n012)
            r = jnp.where(j < n012, cbs[2] + (j - n01), r)
            r = jnp.where(j < n01, cbs[1] + (j - nqs[0]), r)
            return jnp.where(j < nqs[0], cbs[0] + j, r)

        @pl.loop(0, ngrp)
        def _(g):
            for b in range(8):
                j = g * 8 + b

                @pl.when(j < nchl)
                def _(j=j, b=b):
                    jj = _cidx(j)

                    @pl.when(g > 0)
                    def _():
                        pltpu.make_async_copy(
                            gbuf.at[b], acc.at[bdst.at[jj]], ssems[b]).wait()
                    pltpu.async_copy(hc.at[bsrc.at[jj]], gbuf.at[b], gsems[b])
            for b in range(8):
                j = g * 8 + b

                @pl.when(j < nchl)
                def _(j=j, b=b):
                    jj = _cidx(j)
                    pltpu.make_async_copy(
                        hc.at[bsrc.at[jj]], gbuf.at[b], gsems[b]).wait()
                    pltpu.async_copy(gbuf.at[b], acc.at[bdst.at[jj]],
                                     ssems[b], add=True)
        for b in range(8):

            @pl.when(b < nchl)
            def _(b=b):
                pltpu.make_async_copy(
                    gbuf.at[b], acc.at[bdst.at[cbs[0]]], ssems[b]).wait()

        _scope_ed.__exit__(None, None, None)
        plsc.subcore_barrier()

        _scope_ac = jax.named_scope("p2_act")
        _scope_ac.__enter__()

        @pl.when(l < N_BINS - 1)
        def _():
            arow0 = l * LAYER + sid * ACT_PT
            pltpu.sync_copy(acc.at[pl.ds(arow0, ACT_PT)], abuf)

            @pl.loop(0, ACT_PT // 8)
            def _(ch):
                cv = cab[pl.ds(arow0 + ch * 8 + 2, 16)]
                for r in range(8):
                    row = ch * 8 + r
                    code = cv[r]
                    a0 = abuf[row, pl.ds(0, 16)]
                    a1 = abuf[row, pl.ds(16, 16)]
                    abuf[row, pl.ds(0, 16)] = _act_block(a0, code, wv)
                    abuf[row, pl.ds(16, 16)] = _act_block(a1, code, wv)
            pltpu.sync_copy(abuf, hc.at[pl.ds(arow0 + LAYER, ACT_PT)])

        _scope_ac.__exit__(None, None, None)
        plsc.subcore_barrier()

    pltpu.sync_copy(acc.at[pl.ds(LOG0 + sid * 16, 16)],
                    logt.at[cid].at[pl.ds(sid * 16, 16)])


def _softmax_body(lt_ref, o_ref):
    lt = lt_ref[...]
    x = jnp.concatenate(
        [jnp.transpose(lt[0], (1, 0)), jnp.transpose(lt[1], (1, 0))], axis=0)
    m = jnp.max(x, axis=1, keepdims=True)
    e = jnp.exp(x - m)
    o_ref[...] = e / jnp.sum(e, axis=1, keepdims=True)


def kernel(x, weight, edge_src, edge_dst, act_codes):
    x3 = jnp.transpose(x.reshape(NC, HB, INPUT_DIM), (0, 2, 1))
    w16 = jnp.broadcast_to(weight, (LANES,)).astype(jnp.float32)

    logt, _h = _sc_forward(x3, w16, edge_src, edge_dst, act_codes)

    return pl.pallas_call(
        _softmax_body,
        out_shape=jax.ShapeDtypeStruct((BATCH, OUTPUT_DIM), jnp.float32),
    )(logt)


# speedup vs baseline: 1.0668x; 1.0668x over previous
import dataclasses
import functools

import jax
import jax.numpy as jnp
from jax import lax
from jax.experimental import pallas as pl
from jax.experimental.pallas import tpu as pltpu
from jax.experimental.pallas import tpu_sc as plsc

N_NODES = 10000
INPUT_DIM = 512
OUTPUT_DIM = 256
N_LAYERS = 8
LAYER = N_NODES // N_LAYERS
N_EDGES = 160000
BATCH = 64

NC = 2
NS = 16
LANES = 16
HB = BATCH // NC

EPT = N_EDGES // NS
CHUNK = 128
RAW_CHUNKS = -(-EPT // CHUNK)
EPT_PAD = RAW_CHUNKS * CHUNK
RAW_VECS = EPT_PAD // LANES
N_BINS = N_LAYERS - 1
NQ = 2
QUART_VECS = RAW_VECS // NQ
QUART_EDGES = EPT_PAD // NQ
BIN_CHUNKS = -(-(EPT_PAD + NQ * N_BINS * (CHUNK - 1)) // CHUNK) + 1

ACC_REAL = N_NODES - LAYER
ACC_PT = 552
ACC_ROWS = ACC_PT * NS
DUMMY0 = 8752
H_ROWS = 8832
ACT_PT = 80
INIT_PT = 48
LOG0 = ACC_REAL - OUTPUT_DIM

_mesh = plsc.VectorSubcoreMesh(core_axis_name="c", subcore_axis_name="s")

_cp = pltpu.CompilerParams()
for _f, _v in (("needs_layout_passes", False),
               ("use_tc_tiling_on_sc", False)):
    if _f in pltpu.CompilerParams.__dataclass_fields__:
        _cp = dataclasses.replace(_cp, **{_f: _v})


def _key(d):
    return lax.shift_right_logical(d * 6711, 23)


def _act_block(a, code, wv):
    e0 = jnp.exp(-a)
    sig = 1.0 / (1.0 + e0)
    e2 = e0 * e0
    tnh = 2.0 / (1.0 + e2) - 1.0
    rel = jnp.maximum(a, 0.0)
    cb = jnp.full((LANES,), code, dtype=jnp.int32)
    h = jnp.where(cb == 1, sig, a)
    h = jnp.where(cb == 2, rel, h)
    h = jnp.where(cb == 3, tnh, h)
    return h * wv


@functools.partial(
    pl.kernel,
    out_type=[
        jax.ShapeDtypeStruct((NC, OUTPUT_DIM, HB), jnp.float32),
        jax.ShapeDtypeStruct((NC, H_ROWS, HB), jnp.float32),
    ],
    mesh=_mesh,
    scratch_types=[
        pltpu.VMEM_SHARED((ACC_ROWS, HB), jnp.float32),
        pltpu.VMEM((EPT_PAD,), jnp.int32),
        pltpu.VMEM((EPT_PAD,), jnp.int32),
        pltpu.VMEM((BIN_CHUNKS, CHUNK), jnp.int32),
        pltpu.VMEM((BIN_CHUNKS, CHUNK), jnp.int32),
        pltpu.VMEM((8, CHUNK, HB), jnp.float32),
        pltpu.VMEM((ACT_PT, HB), jnp.float32),
        pltpu.VMEM((64, HB), jnp.float32),
        pltpu.VMEM((32, HB), jnp.float32),
        pltpu.VMEM((7552,), jnp.int32),
        pltpu.VMEM((784,), jnp.int32),
        pltpu.VMEM((LANES,), jnp.float32),
        pltpu.VMEM((LANES,), jnp.int32),
        pltpu.VMEM((LANES,), jnp.int32),
        pltpu.VMEM((LANES,), jnp.int32),
        pltpu.VMEM((LANES,), jnp.int32),
        pltpu.SMEM((32,), jnp.int32),
        pltpu.SMEM((32,), jnp.int32),
    ] + [pltpu.SemaphoreType.DMA] * 16,
    compiler_params=_cp,
)
def _sc_forward(x3, w16, esrc, edst, codes, logt, hout,
                acc, rsrc, rdst, bsrc, bdst, gbuf, abuf, zbuf, xbuf,
                cab, cib, wbuf, curvA, curvB, curvC, curvD,
                starts2, nch2, *sems):
    gsems = sems[:8]
    ssems = sems[8:]
    cid = lax.axis_index("c")
    sid = lax.axis_index("s")
    hc = hout.at[cid]

    _scope_p0 = jax.named_scope("p0_stage")
    _scope_p0.__enter__()
    x_src = x3.at[cid].at[pl.ds(sid * 32, 32)]
    s_src = esrc.at[pl.ds(sid * EPT, EPT)]
    d_src = edst.at[pl.ds(sid * EPT, EPT)]
    ca_src = codes.at[pl.ds(LAYER - 2, 7552)]
    ci_src = codes.at[pl.ds(INPUT_DIM, 784)]
    pltpu.async_copy(s_src, rsrc.at[pl.ds(0, EPT)], gsems[0])
    pltpu.async_copy(d_src, rdst.at[pl.ds(0, EPT)], gsems[1])
    pltpu.async_copy(ca_src, cab, gsems[2])
    pltpu.async_copy(ci_src, cib, gsems[3])
    pltpu.async_copy(x_src, xbuf, gsems[4])
    pltpu.sync_copy(w16, wbuf)
    wv = wbuf[...]

    iota = lax.iota(jnp.int32, LANES)

    @pl.loop(0, (EPT_PAD - EPT) // LANES)
    def _(q):
        v = iota + q * LANES
        rsrc[pl.ds(EPT + q * LANES, LANES)] = v & 511
        rdst[pl.ds(EPT + q * LANES, LANES)] = (LAYER + DUMMY0) + (v & 63)

    @pl.loop(0, 64)
    def _(r):
        zbuf[r, pl.ds(0, 16)] = jnp.zeros((16,), jnp.float32)
        zbuf[r, pl.ds(16, 16)] = jnp.zeros((16,), jnp.float32)

    @pl.loop(0, 8)
    def _(k):
        pltpu.sync_copy(zbuf, acc.at[pl.ds(sid * ACC_PT + k * 64, 64)])
    pltpu.sync_copy(zbuf.at[pl.ds(0, 40)],
                    acc.at[pl.ds(sid * ACC_PT + 512, 40)])

    pltpu.make_async_copy(x_src, xbuf, gsems[4]).wait()

    @pl.loop(0, 32)
    def _(r):
        xbuf[r, pl.ds(0, 16)] = xbuf[r, pl.ds(0, 16)] * wv
        xbuf[r, pl.ds(16, 16)] = xbuf[r, pl.ds(16, 16)] * wv
    pltpu.sync_copy(xbuf, hc.at[pl.ds(sid * 32, 32)])

    w_s = wv[0]
    pltpu.make_async_copy(ci_src, cib, gsems[3]).wait()

    @pl.loop(0, INIT_PT // 8)
    def _(ch):
        cv = cib[pl.ds(sid * INIT_PT + ch * 8, 16)]
        for r in range(8):
            row = ch * 8 + r
            val = jnp.where(cv[r] == 1, 0.5 * w_s, 0.0)
            abuf[row, pl.ds(0, 16)] = jnp.full((16,), val, jnp.float32)
            abuf[row, pl.ds(16, 16)] = jnp.full((16,), val, jnp.float32)
    pltpu.sync_copy(abuf.at[pl.ds(0, INIT_PT)],
                    hc.at[pl.ds(INPUT_DIM + sid * INIT_PT, INIT_PT)])

    _scope_p0.__exit__(None, None, None)

    _scope_pf = jax.named_scope("p1_prefill")
    _scope_pf.__enter__()

    @pl.loop(0, BIN_CHUNKS * CHUNK // LANES)
    def _prefill(i):
        jj = i // (CHUNK // LANES)
        qq = i % (CHUNK // LANES)
        v = iota + i * LANES
        bsrc[jj, pl.ds(qq * 16, 16)] = v & 511
        bdst[jj, pl.ds(qq * 16, 16)] = DUMMY0 + (v & 63)

    _scope_pf.__exit__(None, None, None)
    _scope_ct = jax.named_scope("p1_count")
    _scope_ct.__enter__()
    pltpu.make_async_copy(d_src, rdst.at[pl.ds(0, EPT)], gsems[1]).wait()
    pltpu.make_async_copy(s_src, rsrc.at[pl.ds(0, EPT)], gsems[0]).wait()
    pltpu.make_async_copy(ca_src, cab, gsems[2]).wait()
    zv = jnp.zeros((LANES,), jnp.int32)

    def _count_pair(q0):
        def body(i, accs):
            kA = _key(rdst[pl.ds(q0 * QUART_EDGES + i * LANES, LANES)]
                      - LAYER)
            kB = _key(rdst[pl.ds((q0 + 1) * QUART_EDGES + i * LANES, LANES)]
                      - LAYER)
            return (tuple(accs[l] + (kA == l).astype(jnp.int32)
                          for l in range(N_BINS)) +
                    tuple(accs[N_BINS + l] + (kB == l).astype(jnp.int32)
                          for l in range(N_BINS)))
        return lax.fori_loop(0, QUART_VECS, body, (zv,) * (2 * N_BINS))

    accs = _count_pair(0)

    prev = jnp.int32(0)
    for t in range(NQ * N_BINS):
        q, l = t // N_BINS, t % N_BINS
        sl = 8 * q + l
        n = (jnp.sum(accs[t]) + CHUNK - 1) // CHUNK
        nch2[sl] = n
        starts2[sl] = prev
        prev = prev + n * CHUNK

    _scope_ct.__exit__(None, None, None)
    _scope_di = jax.named_scope("p1_dist")
    _scope_di.__enter__()

    curvs = (curvA, curvB)
    for q in range(NQ):
        cur = zv
        for l in range(N_BINS):
            cur = jnp.where(iota == l, starts2[8 * q + l], cur)
        curvs[q][...] = cur

    def _dist_body(i, carry):
        for q in range(NQ):
            off = q * QUART_EDGES + i * LANES
            s = rsrc[pl.ds(off, LANES)]
            dm = rdst[pl.ds(off, LANES)] - LAYER
            k = _key(dm)
            real = k < N_BINS
            cnt, last = plsc.scan_count(k, mask=real)
            base = plsc.load_gather(curvs[q], [k])
            pos = base + cnt - 1
            hi = lax.shift_right_logical(pos, 7)
            lo = pos & (CHUNK - 1)
            plsc.store_scatter(bsrc, [hi, lo], s, mask=real)
            plsc.store_scatter(bdst, [hi, lo], dm, mask=real)
            plsc.addupdate_scatter(curvs[q], [k], cnt, mask=last)
        return carry

    lax.fori_loop(0, QUART_VECS, _dist_body, jnp.int32(0))

    _scope_di.__exit__(None, None, None)

    plsc.subcore_barrier()

    @pl.loop(0, N_BINS)
    def _(l):
        _scope_ed = jax.named_scope("p2_edges")
        _scope_ed.__enter__()
        cbs = [starts2[8 * q + l] // CHUNK for q in range(NQ)]
        nqs = [nch2[8 * q + l] for q in range(NQ)]
        nchl = nqs[0] + nqs[1]
        ngrp = (nchl + 7) // 8

        def _cidx(j):
            return jnp.where(j < nqs[0], cbs[0] + j, cbs[1] + (j - nqs[0]))

        @pl.loop(0, ngrp)
        def _(g):
            for b in range(8):
                j = g * 8 + b

                @pl.when(j < nchl)
                def _(j=j, b=b):
                    jj = _cidx(j)

                    @pl.when(g > 0)
                    def _():
                        pltpu.make_async_copy(
                            gbuf.at[b], acc.at[bdst.at[jj]], ssems[b]).wait()
                    pltpu.async_copy(hc.at[bsrc.at[jj]], gbuf.at[b], gsems[b])
            for b in range(8):
                j = g * 8 + b

                @pl.when(j < nchl)
                def _(j=j, b=b):
                    jj = _cidx(j)
                    pltpu.make_async_copy(
                        hc.at[bsrc.at[jj]], gbuf.at[b], gsems[b]).wait()
                    pltpu.async_copy(gbuf.at[b], acc.at[bdst.at[jj]],
                                     ssems[b], add=True)
        for b in range(8):

            @pl.when(b < nchl)
            def _(b=b):
                pltpu.make_async_copy(
                    gbuf.at[b], acc.at[bdst.at[cbs[0]]], ssems[b]).wait()

        _scope_ed.__exit__(None, None, None)
        plsc.subcore_barrier()

        _scope_ac = jax.named_scope("p2_act")
        _scope_ac.__enter__()

        @pl.when(l < N_BINS - 1)
        def _():
            arow0 = l * LAYER + sid * ACT_PT
            pltpu.sync_copy(acc.at[pl.ds(arow0, ACT_PT)], abuf)

            @pl.loop(0, ACT_PT // 8)
            def _(ch):
                cv = cab[pl.ds(arow0 + ch * 8 + 2, 16)]
                for r in range(8):
                    row = ch * 8 + r
                    code = cv[r]
                    a0 = abuf[row, pl.ds(0, 16)]
                    a1 = abuf[row, pl.ds(16, 16)]
                    abuf[row, pl.ds(0, 16)] = _act_block(a0, code, wv)
                    abuf[row, pl.ds(16, 16)] = _act_block(a1, code, wv)
            pltpu.sync_copy(abuf, hc.at[pl.ds(arow0 + LAYER, ACT_PT)])

        _scope_ac.__exit__(None, None, None)
        plsc.subcore_barrier()

    pltpu.sync_copy(acc.at[pl.ds(LOG0 + sid * 16, 16)],
                    logt.at[cid].at[pl.ds(sid * 16, 16)])


def _softmax_body(lt_ref, o_ref):
    lt = lt_ref[...]
    x = jnp.concatenate(
        [jnp.transpose(lt[0], (1, 0)), jnp.transpose(lt[1], (1, 0))], axis=0)
    m = jnp.max(x, axis=1, keepdims=True)
    e = jnp.exp(x - m)
    o_ref[...] = e / jnp.sum(e, axis=1, keepdims=True)


def kernel(x, weight, edge_src, edge_dst, act_codes):
    x3 = jnp.transpose(x.reshape(NC, HB, INPUT_DIM), (0, 2, 1))
    w16 = jnp.broadcast_to(weight, (LANES,)).astype(jnp.float32)

    logt, _h = _sc_forward(x3, w16, edge_src, edge_dst, act_codes)

    return pl.pallas_call(
        _softmax_body,
        out_shape=jax.ShapeDtypeStruct((BATCH, OUTPUT_DIM), jnp.float32),
    )(logt)
